# baseline (speedup 1.0000x reference)
import functools

import jax
import jax.numpy as jnp
from jax import lax
from jax.experimental import pallas as pl
from jax.experimental.pallas import tpu as pltpu
from jax.experimental.pallas import tpu_sc as plsc

_W_ENERGY = 1.0
_W_FORCE = 0.999

_NC, _NS, _L = 2, 16, 16
_NW = _NC * _NS
_N = 131072
_M = 4096
_APW = _N // _NW
_APS = _N // _NS
_MPW = _M // _NW

_mesh = plsc.VectorSubcoreMesh(core_axis_name="c", subcore_axis_name="s")


@functools.partial(
    pl.kernel,
    out_type=jax.ShapeDtypeStruct((_NC, _L), jnp.float32),
    mesh=_mesh,
    compiler_params=pltpu.CompilerParams(
        needs_layout_passes=False, use_tc_tiling_on_sc=False),
    scratch_types=[
        pltpu.VMEM((_APS,), jnp.int32),
        pltpu.VMEM((_APS,), jnp.float32),
        pltpu.VMEM((_APW * 4,), jnp.float32),
        pltpu.VMEM((_APW * 4,), jnp.float32),
        pltpu.VMEM((_M,), jnp.float32),
        pltpu.VMEM((_MPW,), jnp.float32),
        pltpu.VMEM((_MPW,), jnp.float32),
        pltpu.VMEM((_NS * _L,), jnp.float32),
        pltpu.VMEM((_L,), jnp.float32),
        pltpu.VMEM_SHARED((_M,), jnp.float32),
        pltpu.VMEM_SHARED((_NS * _L,), jnp.float32),
    ],
)
def _loss_sc(ep_h, et_h, fp_h, ft_h, idx_h, out_h,
             idx_v, ones_v, fp_v, ft_v, cnt_v, ep_v, et_v, red_v, tmp_v,
             counts_sh, parts_sh):
    cid = lax.axis_index("c")
    sid = lax.axis_index("s")
    w2 = sid * _NC + cid
    abase = w2 * _APW
    mbase = w2 * _MPW

    pltpu.sync_copy(idx_h.at[pl.ds(sid * _APS, _APS)], idx_v)
    pltpu.sync_copy(fp_h.at[pl.ds(abase * 4, _APW * 4)], fp_v)
    pltpu.sync_copy(ft_h.at[pl.ds(abase * 4, _APW * 4)], ft_v)
    pltpu.sync_copy(ep_h.at[pl.ds(mbase, _MPW)], ep_v)
    pltpu.sync_copy(et_h.at[pl.ds(mbase, _MPW)], et_v)

    ones16 = jnp.ones((_L,), jnp.float32)
    zeros16 = jnp.zeros((_L,), jnp.float32)
    iota = lax.iota(jnp.int32, _L)

    def _fill_ones(i, _):
        ones_v[pl.ds(i * _L, _L)] = ones16
        return 0
    lax.fori_loop(0, _APS // _L, _fill_ones, 0)

    def _fill_zero(i, _):
        red_v[pl.ds(i * _L, _L)] = zeros16
        return 0
    lax.fori_loop(0, _NS, _fill_zero, 0)

    pltpu.sync_copy(red_v, counts_sh.at[pl.ds(sid * (_M // _NS), _M // _NS)])
    plsc.subcore_barrier()
    pltpu.sync_copy(ones_v, counts_sh.at[idx_v], add=True)
    plsc.subcore_barrier()

    pltpu.sync_copy(counts_sh, cnt_v)

    def _invert(i, _):
        c = cnt_v[pl.ds(i * _L, _L)]
        cnt_v[pl.ds(i * _L, _L)] = ones16 / c
        return 0
    lax.fori_loop(0, _M // _L, _invert, 0)

    qiota = iota // 4
    idx_off = cid * _APW

    def _force(i, acc):
        dp = fp_v[pl.ds(i * _L, _L)] - ft_v[pl.ds(i * _L, _L)]
        iv = plsc.load_gather(idx_v, [idx_off + i * 4 + qiota])
        icnt = plsc.load_gather(cnt_v, [iv])
        return acc + dp * dp * icnt
    f_acc = lax.fori_loop(0, _APW * 4 // _L, _force,
                          jnp.zeros((_L,), jnp.float32))

    def _energy(j, acc):
        d = ep_v[pl.ds(j * _L, _L)] - et_v[pl.ds(j * _L, _L)]
        ic = cnt_v[pl.ds(mbase + j * _L, _L)]
        return acc + d * d * ic
    e_acc = lax.fori_loop(0, _MPW // _L, _energy, jnp.zeros((_L,), jnp.float32))

    e_part = jnp.sum(e_acc)
    f_part = jnp.sum(f_acc)
    pv = jnp.where(iota == 0, e_part, jnp.where(iota == 1, f_part, 0.0))
    tmp_v[...] = pv
    pltpu.sync_copy(tmp_v, parts_sh.at[pl.ds(sid * _L, _L)])
    plsc.subcore_barrier()

    @pl.when(sid == 0)
    def _finalize():
        pltpu.sync_copy(parts_sh, red_v)

        def _reduce(s, acc):
            return acc + red_v[pl.ds(s * _L, _L)]
        sums = lax.fori_loop(0, _NS, _reduce, jnp.zeros((_L,), jnp.float32))
        e_b = jnp.sum(jnp.where(iota == 0, sums, 0.0))
        f_b = jnp.sum(jnp.where(iota == 1, sums, 0.0))
        e_loss = e_b * (_W_ENERGY / _M)
        f_loss = f_b * (_W_FORCE / _M)
        tot = e_loss + f_loss
        outv = jnp.where(iota == 0, tot,
                         jnp.where(iota == 1, e_loss,
                                   jnp.where(iota == 2, f_loss, 0.0)))
        tmp_v[...] = outv
        pltpu.sync_copy(tmp_v, out_h.at[cid])


def kernel(per_molecule_energy_predict, per_molecule_energy_true,
           per_atom_force_predict, per_atom_force_true,
           atomic_subsystem_indices):
    out = _loss_sc(
        per_molecule_energy_predict.reshape(_M),
        per_molecule_energy_true.reshape(_M),
        jnp.pad(per_atom_force_predict, ((0, 0), (0, 1))).reshape(_N * 4),
        jnp.pad(per_atom_force_true, ((0, 0), (0, 1))).reshape(_N * 4),
        atomic_subsystem_indices.astype(jnp.int32),
    )
    total = out[0, 0] + out[1, 0]
    e_loss = out[0, 1] + out[1, 1]
    f_loss = out[0, 2] + out[1, 2]
    return (total, e_loss, f_loss)


# speedup vs baseline: 6.9692x; 6.9692x over previous
import functools

import jax
import jax.numpy as jnp
from jax import lax
from jax.experimental import pallas as pl
from jax.experimental.pallas import tpu as pltpu
from jax.experimental.pallas import tpu_sc as plsc

_W_ENERGY = 1.0
_W_FORCE = 0.999

_NC, _NS, _L = 2, 16, 16
_NW = _NC * _NS
_N = 131072
_M = 4096
_APW = _N // _NW
_APS = _N // _NS
_MPW = _M // _NW

_mesh = plsc.VectorSubcoreMesh(core_axis_name="c", subcore_axis_name="s")


@functools.partial(
    pl.kernel,
    out_type=jax.ShapeDtypeStruct((_NC, _L), jnp.float32),
    mesh=_mesh,
    compiler_params=pltpu.CompilerParams(
        needs_layout_passes=False, use_tc_tiling_on_sc=False),
    scratch_types=[
        pltpu.VMEM((_APS,), jnp.int32),
        pltpu.VMEM((_APS,), jnp.float32),
        pltpu.VMEM((_APW,), jnp.float32),
        pltpu.VMEM((_APW,), jnp.float32),
        pltpu.VMEM((_APW,), jnp.float32),
        pltpu.VMEM((_APW,), jnp.float32),
        pltpu.VMEM((_APW,), jnp.float32),
        pltpu.VMEM((_APW,), jnp.float32),
        pltpu.VMEM((_M,), jnp.float32),
        pltpu.VMEM((_MPW,), jnp.float32),
        pltpu.VMEM((_MPW,), jnp.float32),
        pltpu.VMEM((_NS * _L,), jnp.float32),
        pltpu.VMEM((_L,), jnp.float32),
        pltpu.VMEM((_APW,), jnp.float32),
        pltpu.VMEM_SHARED((_M,), jnp.float32),
        pltpu.VMEM_SHARED((_NS * _L,), jnp.float32),
        pltpu.SemaphoreType.DMA,
        pltpu.SemaphoreType.DMA,
    ],
)
def _loss_sc(ep_h, et_h, fpx_h, fpy_h, fpz_h, ftx_h, fty_h, ftz_h, idx_h,
             out_h,
             idx_v, ones_v, fpx_v, fpy_v, fpz_v, ftx_v, fty_v, ftz_v,
             cnt_v, ep_v, et_v, red_v, tmp_v, sq_v,
             counts_sh, parts_sh, sem, sem2):
    cid = lax.axis_index("c")
    sid = lax.axis_index("s")
    w2 = sid * _NC + cid
    abase = w2 * _APW
    mbase = w2 * _MPW

    cp_idx = pltpu.async_copy(idx_h.at[pl.ds(sid * _APS, _APS)], idx_v, sem2)
    cps = [
        pltpu.async_copy(fpx_h.at[pl.ds(abase, _APW)], fpx_v, sem),
        pltpu.async_copy(fpy_h.at[pl.ds(abase, _APW)], fpy_v, sem),
        pltpu.async_copy(fpz_h.at[pl.ds(abase, _APW)], fpz_v, sem),
        pltpu.async_copy(ftx_h.at[pl.ds(abase, _APW)], ftx_v, sem),
        pltpu.async_copy(fty_h.at[pl.ds(abase, _APW)], fty_v, sem),
        pltpu.async_copy(ftz_h.at[pl.ds(abase, _APW)], ftz_v, sem),
        pltpu.async_copy(ep_h.at[pl.ds(mbase, _MPW)], ep_v, sem),
        pltpu.async_copy(et_h.at[pl.ds(mbase, _MPW)], et_v, sem),
    ]

    ones16 = jnp.ones((_L,), jnp.float32)
    zeros16 = jnp.zeros((_L,), jnp.float32)
    iota = lax.iota(jnp.int32, _L)

    @plsc.parallel_loop(0, _APS // _L, unroll=8)
    def _fill_ones(i):
        ones_v[pl.ds(i * _L, _L)] = ones16

    @plsc.parallel_loop(0, _NS, unroll=4)
    def _fill_zero(i):
        red_v[pl.ds(i * _L, _L)] = zeros16

    pltpu.sync_copy(red_v, counts_sh.at[pl.ds(sid * (_M // _NS), _M // _NS)])
    cp_idx.wait()
    plsc.subcore_barrier()
    scatter_cp = pltpu.async_copy(ones_v, counts_sh.at[idx_v], sem2, add=True)
    for cp in cps:
        cp.wait()

    @plsc.parallel_loop(0, _APW // _L, unroll=8)
    def _sq(i):
        a0 = i * _L
        dx = fpx_v[pl.ds(a0, _L)] - ftx_v[pl.ds(a0, _L)]
        dy = fpy_v[pl.ds(a0, _L)] - fty_v[pl.ds(a0, _L)]
        dz = fpz_v[pl.ds(a0, _L)] - ftz_v[pl.ds(a0, _L)]
        sq_v[pl.ds(a0, _L)] = dx * dx + dy * dy + dz * dz

    scatter_cp.wait()
    plsc.subcore_barrier()

    pltpu.sync_copy(counts_sh, cnt_v)

    idx_off = cid * _APW

    @plsc.parallel_loop(0, _APW // _L, unroll=8,
                        carry=jnp.zeros((_L,), jnp.float32))
    def _force(i, acc):
        a0 = i * _L
        iv = idx_v[pl.ds(idx_off + a0, _L)]
        cnt = plsc.load_gather(cnt_v, [iv])
        return acc + sq_v[pl.ds(a0, _L)] / cnt
    f_acc = _force

    @plsc.parallel_loop(0, _MPW // _L, unroll=2,
                        carry=jnp.zeros((_L,), jnp.float32))
    def _energy(j, acc):
        d = ep_v[pl.ds(j * _L, _L)] - et_v[pl.ds(j * _L, _L)]
        c = cnt_v[pl.ds(mbase + j * _L, _L)]
        return acc + d * d / c
    e_acc = _energy

    e_part = jnp.sum(e_acc)
    f_part = jnp.sum(f_acc)
    pv = jnp.where(iota == 0, e_part, jnp.where(iota == 1, f_part, 0.0))
    tmp_v[...] = pv
    pltpu.sync_copy(tmp_v, parts_sh.at[pl.ds(sid * _L, _L)])
    plsc.subcore_barrier()

    @pl.when(sid == 0)
    def _finalize():
        pltpu.sync_copy(parts_sh, red_v)

        def _reduce(s, acc):
            return acc + red_v[pl.ds(s * _L, _L)]
        sums = lax.fori_loop(0, _NS, _reduce, jnp.zeros((_L,), jnp.float32))
        e_b = jnp.sum(jnp.where(iota == 0, sums, 0.0))
        f_b = jnp.sum(jnp.where(iota == 1, sums, 0.0))
        e_loss = e_b * (_W_ENERGY / _M)
        f_loss = f_b * (_W_FORCE / _M)
        tot = e_loss + f_loss
        outv = jnp.where(iota == 0, tot,
                         jnp.where(iota == 1, e_loss,
                                   jnp.where(iota == 2, f_loss, 0.0)))
        tmp_v[...] = outv
        pltpu.sync_copy(tmp_v, out_h.at[cid])


def kernel(per_molecule_energy_predict, per_molecule_energy_true,
           per_atom_force_predict, per_atom_force_true,
           atomic_subsystem_indices):
    fp = per_atom_force_predict
    ft = per_atom_force_true
    out = _loss_sc(
        per_molecule_energy_predict.reshape(_M),
        per_molecule_energy_true.reshape(_M),
        fp[:, 0], fp[:, 1], fp[:, 2],
        ft[:, 0], ft[:, 1], ft[:, 2],
        atomic_subsystem_indices.astype(jnp.int32),
    )
    total = out[0, 0] + out[1, 0]
    e_loss = out[0, 1] + out[1, 1]
    f_loss = out[0, 2] + out[1, 2]
    return (total, e_loss, f_loss)
